# baseline (speedup 1.0000x reference)
import functools

import jax
import jax.numpy as jnp
from jax import lax
from jax.experimental import pallas as pl
from jax.experimental.pallas import tpu as pltpu
from jax.experimental.pallas import tpu_sc as plsc

_NUM_EMBEDDINGS = 1024
_DIM = 64
_ROWS = 9216
_BLK = 1024
_GRID = _ROWS // _BLK
_COMMIT = 0.25
_LOSS_SCALE = _COMMIT / float(_ROWS * _DIM)

_NUM_WORKERS = 32
_ROWS_PER_WORKER = _ROWS // _NUM_WORKERS


def _tc_body(z_ref, e_ref, idx_ref, loss_ref):
    z = z_ref[...]
    e = e_ref[...]
    mm_t = lax.dot_general(e * -2.0, z, (((1,), (1,)), ((), ())),
                           preferred_element_type=jnp.float32)
    ones_row = jnp.ones((1, _DIM), jnp.float32)
    zsq = lax.dot_general(ones_row, z * z, (((1,), (1,)), ((), ())),
                          preferred_element_type=jnp.float32)
    esq = lax.dot_general(e * e, ones_row, (((1,), (1,)), ((), ())),
                          preferred_element_type=jnp.float32)
    dist = (zsq + esq) + mm_t
    dmin = jnp.min(dist, axis=0, keepdims=True)
    ids = lax.broadcasted_iota(jnp.int32, dist.shape, 0).astype(jnp.float32)
    idxf = jnp.min(jnp.where(dist == dmin, ids, 2048.0), axis=0)
    idx_ref[...] = idxf.astype(jnp.int32)

    step = pl.program_id(0)
    prev = loss_ref[...]
    acc = jnp.where(step == 0, 0.0, prev[0, 0]) + jnp.sum(dmin)
    out = jnp.where(step == _GRID - 1, acc * _LOSS_SCALE, acc)
    loss_ref[...] = out.reshape(1, 1)


def _tc_argmin(z_flat, embedding):
    return pl.pallas_call(
        _tc_body,
        grid=(_GRID,),
        in_specs=[
            pl.BlockSpec((_BLK, _DIM), lambda i: (i, 0)),
            pl.BlockSpec((_NUM_EMBEDDINGS, _DIM), lambda i: (0, 0)),
        ],
        out_specs=[
            pl.BlockSpec((_BLK,), lambda i: (i,)),
            pl.BlockSpec((1, 1), lambda i: (0, 0)),
        ],
        out_shape=[
            jax.ShapeDtypeStruct((_ROWS,), jnp.int32),
            jax.ShapeDtypeStruct((1, 1), jnp.float32),
        ],
    )(z_flat, embedding)


def _sc_gather(embedding_padded, idx):
    mesh = plsc.VectorSubcoreMesh(core_axis_name="c", subcore_axis_name="s")

    @functools.partial(
        pl.kernel,
        out_type=jax.ShapeDtypeStruct((_ROWS, 128), jnp.float32),
        mesh=mesh,
        scratch_types=[
            pltpu.VMEM((_ROWS_PER_WORKER,), jnp.int32),
            pltpu.VMEM((_ROWS_PER_WORKER, 128), jnp.float32),
            pltpu.SemaphoreType.DMA,
        ],
    )
    def gather_kernel(e_hbm, idx_hbm, out_hbm, idx_v, rows_v, sem):
        wid = lax.axis_index("s") * 2 + lax.axis_index("c")
        base = wid * _ROWS_PER_WORKER
        pltpu.sync_copy(idx_hbm.at[pl.ds(base, _ROWS_PER_WORKER)], idx_v)
        pltpu.async_copy(e_hbm.at[idx_v], rows_v, sem).wait()
        pltpu.sync_copy(rows_v, out_hbm.at[pl.ds(base, _ROWS_PER_WORKER)])

    return gather_kernel(embedding_padded, idx)


def kernel(inputs, embedding):
    z_flat = inputs.reshape(_ROWS, _DIM)
    idx, loss = _tc_argmin(z_flat, embedding)
    e_pad = jnp.pad(embedding, ((0, 0), (0, 128 - _DIM)))
    quantized = _sc_gather(e_pad, idx)[:, :_DIM]
    return quantized.reshape(inputs.shape), loss[0, 0], idx


# speedup vs baseline: 1.0369x; 1.0369x over previous
import functools

import jax
import jax.numpy as jnp
from jax import lax
from jax.experimental import pallas as pl
from jax.experimental.pallas import tpu as pltpu
from jax.experimental.pallas import tpu_sc as plsc

_NUM_EMBEDDINGS = 1024
_DIM = 64
_ROWS = 9216
_BLK = 3072
_GRID = _ROWS // _BLK
_COMMIT = 0.25
_LOSS_SCALE = _COMMIT / float(_ROWS * _DIM)

_NUM_WORKERS = 32
_ROWS_PER_WORKER = _ROWS // _NUM_WORKERS


def _tc_body(z_ref, e_ref, idx_ref, loss_ref):
    z = z_ref[...]
    e = e_ref[...]
    mm_t = lax.dot_general(e * -2.0, z, (((1,), (1,)), ((), ())),
                           preferred_element_type=jnp.float32)
    ones_row = jnp.ones((1, _DIM), jnp.float32)
    zsq = lax.dot_general(ones_row, z * z, (((1,), (1,)), ((), ())),
                          preferred_element_type=jnp.float32)
    esq = lax.dot_general(e * e, ones_row, (((1,), (1,)), ((), ())),
                          preferred_element_type=jnp.float32)
    dist = (zsq + esq) + mm_t
    dmin = jnp.min(dist, axis=0, keepdims=True)
    ids = lax.broadcasted_iota(jnp.int32, dist.shape, 0).astype(jnp.float32)
    idxf = jnp.min(jnp.where(dist == dmin, ids, 2048.0), axis=0)
    idx_ref[...] = idxf.astype(jnp.int32)

    step = pl.program_id(0)
    prev = loss_ref[...]
    acc = jnp.where(step == 0, 0.0, prev[0, 0]) + jnp.sum(dmin)
    out = jnp.where(step == _GRID - 1, acc * _LOSS_SCALE, acc)
    loss_ref[...] = out.reshape(1, 1)


def _tc_argmin(z_flat, embedding):
    return pl.pallas_call(
        _tc_body,
        grid=(_GRID,),
        in_specs=[
            pl.BlockSpec((_BLK, _DIM), lambda i: (i, 0)),
            pl.BlockSpec((_NUM_EMBEDDINGS, _DIM), lambda i: (0, 0)),
        ],
        out_specs=[
            pl.BlockSpec((_BLK,), lambda i: (i,)),
            pl.BlockSpec((1, 1), lambda i: (0, 0)),
        ],
        out_shape=[
            jax.ShapeDtypeStruct((_ROWS,), jnp.int32),
            jax.ShapeDtypeStruct((1, 1), jnp.float32),
        ],
    )(z_flat, embedding)


def _sc_gather(embedding, idx):
    mesh = plsc.VectorSubcoreMesh(core_axis_name="c", subcore_axis_name="s")

    @functools.partial(
        pl.kernel,
        out_type=jax.ShapeDtypeStruct((_ROWS, _DIM), jnp.float32),
        mesh=mesh,
        scratch_types=[
            pltpu.VMEM((_ROWS_PER_WORKER,), jnp.int32),
            pltpu.VMEM((_ROWS_PER_WORKER, _DIM), jnp.float32),
            pltpu.SemaphoreType.DMA,
        ],
        compiler_params=pltpu.CompilerParams(use_tc_tiling_on_sc=False),
    )
    def gather_kernel(e_hbm, idx_hbm, out_hbm, idx_v, rows_v, sem):
        wid = lax.axis_index("s") * 2 + lax.axis_index("c")
        base = wid * _ROWS_PER_WORKER
        pltpu.sync_copy(idx_hbm.at[pl.ds(base, _ROWS_PER_WORKER)], idx_v)
        pltpu.async_copy(e_hbm.at[idx_v], rows_v, sem).wait()
        pltpu.sync_copy(rows_v, out_hbm.at[pl.ds(base, _ROWS_PER_WORKER)])

    return gather_kernel(embedding, idx)


def kernel(inputs, embedding):
    z_flat = inputs.reshape(_ROWS, _DIM)
    idx, loss = _tc_argmin(z_flat, embedding)
    quantized = _sc_gather(embedding, idx)
    return quantized.reshape(inputs.shape), loss[0, 0], idx


# speedup vs baseline: 1.1980x; 1.1554x over previous
import functools

import jax
import jax.numpy as jnp
from jax import lax
from jax.experimental import pallas as pl
from jax.experimental.pallas import tpu as pltpu
from jax.experimental.pallas import tpu_sc as plsc

_NUM_EMBEDDINGS = 1024
_DIM = 64
_ROWS = 9216
_BLK = 1024
_GRID = _ROWS // _BLK
_COMMIT = 0.25
_LOSS_SCALE = _COMMIT / float(_ROWS * _DIM)
_GCODES = 256
_NGROUPS = _NUM_EMBEDDINGS // _GCODES

_NUM_WORKERS = 32
_ROWS_PER_WORKER = _ROWS // _NUM_WORKERS


def _tc_body(z_ref, e_ref, idx_ref, loss_ref):
    z = z_ref[...]
    e = e_ref[...]
    ones_row = jnp.ones((1, _DIM), jnp.float32)
    zsq = lax.dot_general(ones_row, z * z, (((1,), (1,)), ((), ())),
                          preferred_element_type=jnp.float32)
    esq = lax.dot_general(e * e, ones_row, (((1,), (1,)), ((), ())),
                          preferred_element_type=jnp.float32)
    e2 = e * -2.0
    ids8 = lax.broadcasted_iota(jnp.int32, (8, _BLK), 0)
    best_d = None
    best_i = None
    for g in range(_NGROUPS):
        lo = g * _GCODES
        mm_g = lax.dot_general(e2[lo:lo + _GCODES], z, (((1,), (1,)), ((), ())),
                               preferred_element_type=jnp.float32)
        for c in range(0, _GCODES, 8):
            d_c = (zsq + esq[lo + c:lo + c + 8]) + mm_g[c:c + 8]
            i_c = ids8 + (lo + c)
            if best_d is None:
                best_d, best_i = d_c, i_c
            else:
                m = d_c < best_d
                best_d = jnp.where(m, d_c, best_d)
                best_i = jnp.where(m, i_c, best_i)
    for sh in (4, 2, 1):
        a_d, b_d = best_d[:sh], best_d[sh:2 * sh]
        a_i, b_i = best_i[:sh], best_i[sh:2 * sh]
        m = (b_d < a_d) | ((b_d == a_d) & (b_i < a_i))
        best_i = jnp.where(m, b_i, a_i)
        best_d = jnp.where(m, b_d, a_d)
    idx_ref[...] = best_i.reshape(_BLK)

    step = pl.program_id(0)
    prev = loss_ref[...]
    acc = jnp.where(step == 0, 0.0, prev[0, 0]) + jnp.sum(best_d)
    out = jnp.where(step == _GRID - 1, acc * _LOSS_SCALE, acc)
    loss_ref[...] = out.reshape(1, 1)


def _tc_argmin(z_flat, embedding):
    return pl.pallas_call(
        _tc_body,
        grid=(_GRID,),
        in_specs=[
            pl.BlockSpec((_BLK, _DIM), lambda i: (i, 0)),
            pl.BlockSpec((_NUM_EMBEDDINGS, _DIM), lambda i: (0, 0)),
        ],
        out_specs=[
            pl.BlockSpec((_BLK,), lambda i: (i,)),
            pl.BlockSpec((1, 1), lambda i: (0, 0)),
        ],
        out_shape=[
            jax.ShapeDtypeStruct((_ROWS,), jnp.int32),
            jax.ShapeDtypeStruct((1, 1), jnp.float32),
        ],
    )(z_flat, embedding)


def _sc_gather(embedding, idx):
    mesh = plsc.VectorSubcoreMesh(core_axis_name="c", subcore_axis_name="s")

    @functools.partial(
        pl.kernel,
        out_type=jax.ShapeDtypeStruct((_ROWS, _DIM), jnp.float32),
        mesh=mesh,
        scratch_types=[
            pltpu.VMEM((_ROWS_PER_WORKER,), jnp.int32),
            pltpu.VMEM((_ROWS_PER_WORKER, _DIM), jnp.float32),
            pltpu.SemaphoreType.DMA,
        ],
        compiler_params=pltpu.CompilerParams(use_tc_tiling_on_sc=False),
    )
    def gather_kernel(e_hbm, idx_hbm, out_hbm, idx_v, rows_v, sem):
        wid = lax.axis_index("s") * 2 + lax.axis_index("c")
        base = wid * _ROWS_PER_WORKER
        pltpu.sync_copy(idx_hbm.at[pl.ds(base, _ROWS_PER_WORKER)], idx_v)
        pltpu.async_copy(e_hbm.at[idx_v], rows_v, sem).wait()
        pltpu.sync_copy(rows_v, out_hbm.at[pl.ds(base, _ROWS_PER_WORKER)])

    return gather_kernel(embedding, idx)


def kernel(inputs, embedding):
    z_flat = inputs.reshape(_ROWS, _DIM)
    idx, loss = _tc_argmin(z_flat, embedding)
    quantized = _sc_gather(embedding, idx)
    return quantized.reshape(inputs.shape), loss[0, 0], idx


# speedup vs baseline: 1.2041x; 1.0051x over previous
import functools

import jax
import jax.numpy as jnp
from jax import lax
from jax.experimental import pallas as pl
from jax.experimental.pallas import tpu as pltpu
from jax.experimental.pallas import tpu_sc as plsc

_NUM_EMBEDDINGS = 1024
_DIM = 64
_ROWS = 9216
_BLK = 1024
_GRID = _ROWS // _BLK
_COMMIT = 0.25
_LOSS_SCALE = _COMMIT / float(_ROWS * _DIM)
_GCODES = 256
_NGROUPS = _NUM_EMBEDDINGS // _GCODES

_NUM_WORKERS = 32
_ROWS_PER_WORKER = _ROWS // _NUM_WORKERS


def _tc_body(z_ref, e_ref, idx_ref, loss_ref):
    z = z_ref[...]
    e = e_ref[...]
    ones_row = jnp.ones((1, _DIM), jnp.float32)
    zsq = lax.dot_general(ones_row, z * z, (((1,), (1,)), ((), ())),
                          preferred_element_type=jnp.float32)
    esq = lax.dot_general(e * e, ones_row, (((1,), (1,)), ((), ())),
                          preferred_element_type=jnp.float32)
    e2 = e * -2.0
    ids8 = lax.broadcasted_iota(jnp.int32, (8, _BLK), 0)
    mm = lax.dot_general(e2, z, (((1,), (1,)), ((), ())),
                         preferred_element_type=jnp.float32)
    best_d = None
    best_i = None
    for c in range(0, _NUM_EMBEDDINGS, 8):
        d_c = (zsq + esq[c:c + 8]) + mm[c:c + 8]
        i_c = ids8 + c
        if best_d is None:
            best_d, best_i = d_c, i_c
        else:
            m = d_c < best_d
            best_d = jnp.where(m, d_c, best_d)
            best_i = jnp.where(m, i_c, best_i)
    for sh in (4, 2, 1):
        a_d, b_d = best_d[:sh], best_d[sh:2 * sh]
        a_i, b_i = best_i[:sh], best_i[sh:2 * sh]
        m = (b_d < a_d) | ((b_d == a_d) & (b_i < a_i))
        best_i = jnp.where(m, b_i, a_i)
        best_d = jnp.where(m, b_d, a_d)
    idx_ref[...] = best_i.reshape(_BLK)

    step = pl.program_id(0)
    prev = loss_ref[...]
    acc = jnp.where(step == 0, 0.0, prev[0, 0]) + jnp.sum(best_d)
    out = jnp.where(step == _GRID - 1, acc * _LOSS_SCALE, acc)
    loss_ref[...] = out.reshape(1, 1)


def _tc_argmin(z_flat, embedding):
    return pl.pallas_call(
        _tc_body,
        grid=(_GRID,),
        in_specs=[
            pl.BlockSpec((_BLK, _DIM), lambda i: (i, 0)),
            pl.BlockSpec((_NUM_EMBEDDINGS, _DIM), lambda i: (0, 0)),
        ],
        out_specs=[
            pl.BlockSpec((_BLK,), lambda i: (i,)),
            pl.BlockSpec((1, 1), lambda i: (0, 0)),
        ],
        out_shape=[
            jax.ShapeDtypeStruct((_ROWS,), jnp.int32),
            jax.ShapeDtypeStruct((1, 1), jnp.float32),
        ],
    )(z_flat, embedding)


def _sc_gather(embedding, idx):
    mesh = plsc.VectorSubcoreMesh(core_axis_name="c", subcore_axis_name="s")

    @functools.partial(
        pl.kernel,
        out_type=jax.ShapeDtypeStruct((_ROWS, _DIM), jnp.float32),
        mesh=mesh,
        scratch_types=[
            pltpu.VMEM((_ROWS_PER_WORKER,), jnp.int32),
            pltpu.VMEM((_ROWS_PER_WORKER, _DIM), jnp.float32),
            pltpu.SemaphoreType.DMA,
        ],
        compiler_params=pltpu.CompilerParams(use_tc_tiling_on_sc=False),
    )
    def gather_kernel(e_hbm, idx_hbm, out_hbm, idx_v, rows_v, sem):
        wid = lax.axis_index("s") * 2 + lax.axis_index("c")
        base = wid * _ROWS_PER_WORKER
        pltpu.sync_copy(idx_hbm.at[pl.ds(base, _ROWS_PER_WORKER)], idx_v)
        pltpu.async_copy(e_hbm.at[idx_v], rows_v, sem).wait()
        pltpu.sync_copy(rows_v, out_hbm.at[pl.ds(base, _ROWS_PER_WORKER)])

    return gather_kernel(embedding, idx)


def kernel(inputs, embedding):
    z_flat = inputs.reshape(_ROWS, _DIM)
    idx, loss = _tc_argmin(z_flat, embedding)
    quantized = _sc_gather(embedding, idx)
    return quantized.reshape(inputs.shape), loss[0, 0], idx
